# baseline (speedup 1.0000x reference)
import jax
import jax.numpy as jnp
from jax.experimental import pallas as pl
from jax.experimental.pallas import tpu as pltpu

_N_BEHAVIORS = 8
_BB = 64


def _body(pb_ref, thr_ref, beh_ref, pad_ref, s_ref, hp_ref, hn_ref):
    beh = beh_ref[...]
    pad = pad_ref[...]
    thr = thr_ref[...]
    t = 1.0 / (1.0 + jnp.exp(-thr))
    idx = jnp.maximum(beh - 1, 0)
    pref = jnp.zeros_like(pad)
    for k in range(_N_BEHAVIORS):
        pref = jnp.where(idx == k, pb_ref[0, k], pref)
    signal = pref - t
    m = (signal > 0).astype(jnp.float32) * pad
    s = s_ref[...]
    hp_ref[...] = s * m[:, :, None]
    hn_ref[...] = s * ((1.0 - m) * pad)[:, :, None]


def kernel(S, behavior_seq, padding_mask, lambda_raw, threshold):
    B, L, D = S.shape
    lam = jax.nn.softplus(lambda_raw) + 1e-6
    log_pmf = -lam + lam * jnp.log(lam) - jax.lax.lgamma(lam + 1.0)
    p_b = (jnp.exp(log_pmf) + 1.0).reshape(1, _N_BEHAVIORS)
    thr2 = threshold[:L].reshape(1, L)

    grid = (B // _BB,)
    out = pl.pallas_call(
        _body,
        grid=grid,
        in_specs=[
            pl.BlockSpec(memory_space=pltpu.SMEM),
            pl.BlockSpec((1, L), lambda i: (0, 0)),
            pl.BlockSpec((_BB, L), lambda i: (i, 0)),
            pl.BlockSpec((_BB, L), lambda i: (i, 0)),
            pl.BlockSpec((_BB, L, D), lambda i: (i, 0, 0)),
        ],
        out_specs=[
            pl.BlockSpec((_BB, L, D), lambda i: (i, 0, 0)),
            pl.BlockSpec((_BB, L, D), lambda i: (i, 0, 0)),
        ],
        out_shape=[
            jax.ShapeDtypeStruct((B, L, D), jnp.float32),
            jax.ShapeDtypeStruct((B, L, D), jnp.float32),
        ],
    )(p_b, thr2, behavior_seq, padding_mask, S)
    return (out[0], out[1])


# speedup vs baseline: 1.6189x; 1.6189x over previous
import jax
import jax.numpy as jnp
from jax.experimental import pallas as pl
from jax.experimental.pallas import tpu as pltpu

_N_BEHAVIORS = 8
_BB = 64


def _body(pb_ref, thr_ref, beh_ref, pad_ref, e_ref, s_ref, hp_ref, hn_ref):
    beh = beh_ref[...]
    pad = pad_ref[...]
    thr = thr_ref[...]
    t = 1.0 / (1.0 + jnp.exp(-thr))
    idx = jnp.maximum(beh - 1, 0)
    pref = jnp.zeros_like(pad)
    for k in range(_N_BEHAVIORS):
        pref = jnp.where(idx == k, pb_ref[0, k], pref)
    signal = pref - t
    m = (signal > 0).astype(jnp.float32) * pad
    hpf = m.astype(jnp.bfloat16)
    hnf = ((1.0 - m) * pad).astype(jnp.bfloat16)
    e = e_ref[...]
    dn = (((1,), (0,)), ((), ()))
    hp_full = jax.lax.dot_general(hpf, e, dn, preferred_element_type=jnp.float32)
    hn_full = jax.lax.dot_general(hnf, e, dn, preferred_element_type=jnp.float32)
    s = s_ref[...]
    hp_ref[...] = s * hp_full
    hn_ref[...] = s * hn_full


def kernel(S, behavior_seq, padding_mask, lambda_raw, threshold):
    B, L, D = S.shape
    LD = L * D
    lam = jax.nn.softplus(lambda_raw) + 1e-6
    log_pmf = -lam + lam * jnp.log(lam) - jax.lax.lgamma(lam + 1.0)
    p_b = (jnp.exp(log_pmf) + 1.0).reshape(1, _N_BEHAVIORS)
    thr2 = threshold[:L].reshape(1, L)
    expand = (jnp.arange(LD, dtype=jnp.int32)[None, :] // D
              == jnp.arange(L, dtype=jnp.int32)[:, None]).astype(jnp.bfloat16)
    s2 = S.reshape(B, LD)

    grid = (B // _BB,)
    out = pl.pallas_call(
        _body,
        grid=grid,
        in_specs=[
            pl.BlockSpec(memory_space=pltpu.SMEM),
            pl.BlockSpec((1, L), lambda i: (0, 0)),
            pl.BlockSpec((_BB, L), lambda i: (i, 0)),
            pl.BlockSpec((_BB, L), lambda i: (i, 0)),
            pl.BlockSpec((L, LD), lambda i: (0, 0)),
            pl.BlockSpec((_BB, LD), lambda i: (i, 0)),
        ],
        out_specs=[
            pl.BlockSpec((_BB, LD), lambda i: (i, 0)),
            pl.BlockSpec((_BB, LD), lambda i: (i, 0)),
        ],
        out_shape=[
            jax.ShapeDtypeStruct((B, LD), jnp.float32),
            jax.ShapeDtypeStruct((B, LD), jnp.float32),
        ],
    )(p_b, thr2, behavior_seq, padding_mask, expand, s2)
    return (out[0].reshape(B, L, D), out[1].reshape(B, L, D))


# speedup vs baseline: 6.3515x; 3.9234x over previous
import jax
import jax.numpy as jnp
from jax.experimental import pallas as pl
from jax.experimental.pallas import tpu as pltpu

_N_BEHAVIORS = 8
_LB = 8


def _body(pb_ref, thr_ref, beh_ref, pad_ref, s_ref, hp_ref, hn_ref):
    beh = beh_ref[...]
    pad = pad_ref[...]
    i = pl.program_id(0)
    thr = thr_ref[pl.ds(i * _LB, _LB), :]
    t = 1.0 / (1.0 + jnp.exp(-thr))
    idx = jnp.maximum(beh - 1, 0)
    pref = jnp.zeros_like(pad)
    for k in range(_N_BEHAVIORS):
        pref = jnp.where(idx == k, pb_ref[0, k], pref)
    signal = pref - t
    m = (signal > 0).astype(jnp.float32) * pad
    hnf = (1.0 - m) * pad
    s = s_ref[...]
    hp_ref[...] = s * m[:, None, :]
    hn_ref[...] = s * hnf[:, None, :]


def kernel(S, behavior_seq, padding_mask, lambda_raw, threshold):
    B, L, D = S.shape
    lam = jax.nn.softplus(lambda_raw) + 1e-6
    log_pmf = -lam + lam * jnp.log(lam) - jax.lax.lgamma(lam + 1.0)
    p_b = (jnp.exp(log_pmf) + 1.0).reshape(1, _N_BEHAVIORS)
    thr2 = threshold[:L].reshape(L, 1)
    s_t = jnp.transpose(S, (1, 2, 0))
    beh_t = behavior_seq.T
    pad_t = padding_mask.T

    grid = (L // _LB,)
    out = pl.pallas_call(
        _body,
        grid=grid,
        in_specs=[
            pl.BlockSpec(memory_space=pltpu.SMEM),
            pl.BlockSpec((L, 1), lambda i: (0, 0)),
            pl.BlockSpec((_LB, B), lambda i: (i, 0)),
            pl.BlockSpec((_LB, B), lambda i: (i, 0)),
            pl.BlockSpec((_LB, D, B), lambda i: (i, 0, 0)),
        ],
        out_specs=[
            pl.BlockSpec((_LB, D, B), lambda i: (i, 0, 0)),
            pl.BlockSpec((_LB, D, B), lambda i: (i, 0, 0)),
        ],
        out_shape=[
            jax.ShapeDtypeStruct((L, D, B), jnp.float32),
            jax.ShapeDtypeStruct((L, D, B), jnp.float32),
        ],
    )(p_b, thr2, beh_t, pad_t, s_t)
    return (jnp.transpose(out[0], (2, 0, 1)), jnp.transpose(out[1], (2, 0, 1)))
